# Initial kernel scaffold; baseline (speedup 1.0000x reference)
import jax
import jax.numpy as jnp
from jax.experimental import pallas as pl


def kernel(batch_data, user_emb, item_emb, edge_src, edge_dst, behavior_weight, W1, b1, Wmu, bmu, Wlv, blv, W3, b3, W4, b4, eps):
    raise NotImplementedError("write your pallas kernel here")


# speedup vs baseline: 1.0001x; 1.0001x over previous
import jax
import jax.numpy as jnp
from jax.experimental import pallas as pl

N_USERS = 60000
N_ITEMS = 40000
D = 64
LAYERS = 2
N_GRAPHS = 4
N_BEH = 3
B = 4096
BN = 512
N_NODES = (N_USERS + 1) + (N_ITEMS + 1)
KL_REG, ORT_REG, LOG_REG, NCE_REG, BPR_REG, REG_W, TEMP = 0.1, 0.1, 1.0, 0.1, 1.0, 1e-3, 0.2


def _lightgcn(x, src, dst):
    ones = jnp.ones(src.shape[0], dtype=x.dtype)
    deg = jax.ops.segment_sum(ones, dst, num_segments=N_NODES)
    deg = jnp.maximum(deg, 1.0)
    norm = 1.0 / jnp.sqrt(deg[src] * deg[dst])
    acc = x
    h = x
    for _ in range(LAYERS):
        h = jax.ops.segment_sum(h[src] * norm[:, None], dst, num_segments=N_NODES)
        acc = acc + h
    return acc * (1.0 / (LAYERS + 1))


def _loss_kernel(train_u_ref, inv_p_ref, inv_n_ref, item_e_ref, eps_ref,
                 W1_ref, b1_ref, Wmu_ref, bmu_ref, Wlv_ref, blv_ref,
                 W3_ref, b3_ref, W4_ref, b4_ref, out_ref):
    train_u2 = train_u_ref[...].reshape(BN * N_GRAPHS, D)
    h = jnp.maximum(jnp.dot(train_u2, W1_ref[...],
                            preferred_element_type=jnp.float32) + b1_ref[...][None, :], 0.0)
    mu = jnp.dot(h, Wmu_ref[...], preferred_element_type=jnp.float32) + bmu_ref[...][None, :]
    logvar = jnp.dot(h, Wlv_ref[...], preferred_element_type=jnp.float32) + blv_ref[...][None, :]
    eps2 = eps_ref[...].reshape(BN * N_GRAPHS, -1)
    z = mu + jnp.exp(0.5 * logvar) * eps2
    hd = jnp.maximum(jnp.dot(z, W3_ref[...],
                             preferred_element_type=jnp.float32) + b3_ref[...][None, :], 0.0)
    inv2 = jnp.dot(hd, W4_ref[...], preferred_element_type=jnp.float32) + b4_ref[...][None, :]

    kl_part = -0.5 * jnp.sum(1.0 + logvar - mu * mu - jnp.exp(logvar)) / (B * N_GRAPHS * 16)

    var2 = train_u2 - inv2
    inv3 = inv2.reshape(BN, N_GRAPHS, D)
    var3 = var2.reshape(BN, N_GRAPHS, D)

    inv_g = [inv3[:, g, :] for g in range(N_GRAPHS)]
    var_g = [var3[:, g, :] for g in range(N_GRAPHS)]
    zi_g, zv_g = [], []
    for g in range(N_GRAPHS):
        ni = jnp.sqrt(jnp.sum(inv_g[g] * inv_g[g], axis=-1, keepdims=True)) + 1e-8
        nv = jnp.sqrt(jnp.sum(var_g[g] * var_g[g], axis=-1, keepdims=True)) + 1e-8
        zi_g.append(inv_g[g] / ni)
        zv_g.append(var_g[g] / nv)

    nce_sum = 0.0
    for g in range(N_GRAPHS):
        pos = jnp.zeros((BN, 1), jnp.float32)
        neg = jnp.zeros((BN, 1), jnp.float32)
        for hh in range(N_GRAPHS):
            s_ii = jnp.sum(zi_g[g] * zi_g[hh], axis=-1, keepdims=True) / TEMP
            s_iv = jnp.sum(zi_g[g] * zv_g[hh], axis=-1, keepdims=True) / TEMP
            if hh != g:
                pos = pos + jnp.exp(s_ii)
            neg = neg + jnp.exp(s_iv)
        nce_sum = nce_sum + jnp.sum(-jnp.log(pos / (pos + neg) + 1e-12))
    nce_part = nce_sum / (B * N_GRAPHS)

    ort_sum = 0.0
    for g in range(N_GRAPHS):
        s = jnp.sum(inv_g[g] * var_g[g], axis=-1, keepdims=True)
        ort_sum = ort_sum + jnp.sum(s * s)
    ort_part = ort_sum / (B * N_GRAPHS)

    inv_p = inv_p_ref[...]
    log_sum = 0.0
    for g in range(N_GRAPHS):
        sc = jnp.sum(inv_g[g] * inv_p, axis=-1, keepdims=True)
        p = jnp.clip(jax.nn.sigmoid(sc), 1e-7, 1.0 - 1e-7)
        log_sum = log_sum + jnp.sum(-jnp.log(p))
        for j in range(2):
            scn = jnp.sum(inv_g[g] * inv_n_ref[:, j, :], axis=-1, keepdims=True)
            pn = jnp.clip(jax.nn.sigmoid(scn), 1e-7, 1.0 - 1e-7)
            log_sum = log_sum + jnp.sum(-jnp.log(1.0 - pn))
    log_part = log_sum / (B * N_GRAPHS * 3)

    inv_user = (inv_g[0] + inv_g[1] + inv_g[2] + inv_g[3]) * (1.0 / N_GRAPHS)
    var_user = var_g[0]
    sc0 = (jnp.sum(inv_user * item_e_ref[:, 0, :], axis=-1, keepdims=True)
           + jnp.sum(var_user * item_e_ref[:, 0, :], axis=-1, keepdims=True))
    sc1 = (jnp.sum(inv_user * item_e_ref[:, 1, :], axis=-1, keepdims=True)
           + jnp.sum(var_user * item_e_ref[:, 1, :], axis=-1, keepdims=True))
    bpr_part = jnp.sum(-jax.nn.log_sigmoid(sc0 - sc1)) / B

    part = (KL_REG * kl_part + ORT_REG * ort_part + LOG_REG * log_part
            + NCE_REG * nce_part + BPR_REG * bpr_part)
    part = jnp.broadcast_to(part, (1, 1))

    i = pl.program_id(0)

    @pl.when(i == 0)
    def _init():
        out_ref[...] = part

    @pl.when(i != 0)
    def _acc():
        out_ref[...] = out_ref[...] + part


@jax.jit
def kernel(batch_data, user_emb, item_emb, edge_src, edge_dst, behavior_weight,
           W1, b1, Wmu, bmu, Wlv, blv, W3, b3, W4, b4, eps):
    ini = jnp.concatenate([user_emb, item_emb], axis=0)
    pt = _lightgcn(ini, edge_src[-1], edge_dst[-1])
    user_embs, item_embs = [], []
    for g in range(N_GRAPHS):
        be = _lightgcn(pt, edge_src[g], edge_dst[g])
        user_embs.append(be[: N_USERS + 1])
        if g < N_BEH:
            item_embs.append(be[N_USERS + 1:])
    agg_item = jnp.sum(jnp.stack(item_embs, 0) * behavior_weight, axis=0)

    uidx = batch_data[:, 0, 0]
    train_u = jnp.stack([ue[uidx] for ue in user_embs], axis=1)
    inv_p = agg_item[batch_data[:, 0, 1]]
    inv_n = agg_item[batch_data[:, 0, 2:]]
    item_e = agg_item[batch_data[:, -1, 1:3]]

    grid = (B // BN,)
    full2 = lambda shape: pl.BlockSpec(shape, lambda i: (0,) * len(shape))
    out = pl.pallas_call(
        _loss_kernel,
        grid=grid,
        in_specs=[
            pl.BlockSpec((BN, N_GRAPHS, D), lambda i: (i, 0, 0)),
            pl.BlockSpec((BN, D), lambda i: (i, 0)),
            pl.BlockSpec((BN, 2, D), lambda i: (i, 0, 0)),
            pl.BlockSpec((BN, 2, D), lambda i: (i, 0, 0)),
            pl.BlockSpec((BN, N_GRAPHS, 16), lambda i: (i, 0, 0)),
            full2((D, 32)), full2((32,)), full2((32, 16)), full2((16,)),
            full2((32, 16)), full2((16,)), full2((16, 32)), full2((32,)),
            full2((32, D)), full2((D,)),
        ],
        out_specs=pl.BlockSpec((1, 1), lambda i: (0, 0)),
        out_shape=jax.ShapeDtypeStruct((1, 1), jnp.float32),
    )(train_u, inv_p, inv_n, item_e, eps,
      W1, b1, Wmu, bmu, Wlv, blv, W3, b3, W4, b4)

    emb_loss = (jnp.sum(user_emb ** 2) + jnp.sum(item_emb ** 2)) / N_NODES
    return out[0, 0] + REG_W * emb_loss
